# baseline (speedup 1.0000x reference)
import functools

import jax
import jax.numpy as jnp
from jax import lax
from jax.experimental import pallas as pl
from jax.experimental.pallas import tpu as pltpu
from jax.experimental.pallas import tpu_sc as plsc

D = 64
SCALE = 8.0
NC = 2
NS = 16
NW = NC * NS
CHUNK = 128
B = 4096 * 200
BPW = B // NW
NCHUNK = BPW // CHUNK

_mesh = plsc.VectorSubcoreMesh(core_axis_name="c", subcore_axis_name="s")


@functools.partial(
    pl.kernel,
    out_type=jax.ShapeDtypeStruct((B, D), jnp.float32),
    mesh=_mesh,
    scratch_types=[
        pltpu.VMEM((NCHUNK, CHUNK), jnp.int32),
        pltpu.VMEM((CHUNK, D), jnp.float32),
        pltpu.SemaphoreType.DMA,
    ],
    compiler_params=pltpu.CompilerParams(use_tc_tiling_on_sc=False),
)
def _emb_lookup(table_hbm, idx_hbm, out_hbm, idx_v, rows_v, gsem):
    wid = lax.axis_index("s") * NC + lax.axis_index("c")
    base = wid * BPW
    pltpu.sync_copy(idx_hbm.at[wid], idx_v)

    def chunk_body(j, carry):
        pltpu.async_copy(table_hbm.at[idx_v.at[j]], rows_v, gsem).wait()

        def scale_row(i, c):
            for k in range(D // 16):
                sl = pl.ds(k * 16, 16)
                rows_v[i, sl] = rows_v[i, sl] * SCALE
            return c

        lax.fori_loop(0, CHUNK, scale_row, 0, unroll=2)
        pltpu.sync_copy(rows_v, out_hbm.at[pl.ds(base + j * CHUNK, CHUNK)])
        return carry

    lax.fori_loop(0, NCHUNK, chunk_body, 0)


def kernel(x, table):
    idx = x.astype(jnp.int32).reshape(NW, NCHUNK, CHUNK)
    out = _emb_lookup(table, idx)
    return out.reshape(x.shape[0], x.shape[1], D)


# speedup vs baseline: 1.1557x; 1.1557x over previous
import functools

import jax
import jax.numpy as jnp
from jax import lax
from jax.experimental import pallas as pl
from jax.experimental.pallas import tpu as pltpu
from jax.experimental.pallas import tpu_sc as plsc

D = 64
SCALE = 8.0
NC = 2
NS = 16
NW = NC * NS
CHUNK = 128
B = 4096 * 200
BPW = B // NW
NCHUNK = BPW // CHUNK

_mesh = plsc.VectorSubcoreMesh(core_axis_name="c", subcore_axis_name="s")
NBUF = 4
NROUND = NCHUNK // NBUF


@functools.partial(
    pl.kernel,
    out_type=jax.ShapeDtypeStruct((B, D), jnp.float32),
    mesh=_mesh,
    scratch_types=[
        pltpu.VMEM((NCHUNK, CHUNK), jnp.int32),
        pltpu.VMEM((NBUF, CHUNK, D), jnp.float32),
        pltpu.SemaphoreType.DMA((NBUF,)),
        pltpu.SemaphoreType.DMA((NBUF,)),
    ],
    compiler_params=pltpu.CompilerParams(use_tc_tiling_on_sc=False),
)
def _emb_lookup(table_hbm, idx_hbm, out_hbm, idx_v, rows_v, gsem, ssem):
    wid = lax.axis_index("s") * NC + lax.axis_index("c")
    base = wid * BPW
    pltpu.sync_copy(idx_hbm.at[wid], idx_v)

    def gfire(j, b):
        return pltpu.async_copy(table_hbm.at[idx_v.at[j]], rows_v.at[b],
                                gsem.at[b])

    def sfire(j, b):
        return pltpu.async_copy(rows_v.at[b],
                                out_hbm.at[pl.ds(base + j * CHUNK, CHUNK)],
                                ssem.at[b])

    def gwait(j, b):
        pltpu.make_async_copy(table_hbm.at[idx_v.at[j]], rows_v.at[b],
                              gsem.at[b]).wait()

    def swait(j, b):
        pltpu.make_async_copy(rows_v.at[b],
                              out_hbm.at[pl.ds(base + j * CHUNK, CHUNK)],
                              ssem.at[b]).wait()

    def scale(b):
        def scale_row(i, c):
            for k in range(D // 16):
                sl = pl.ds(k * 16, 16)
                rows_v[b, i, sl] = rows_v[b, i, sl] * SCALE
            return c

        lax.fori_loop(0, CHUNK, scale_row, 0, unroll=4)

    for b in range(NBUF):
        gfire(b, b)

    def round_body(g, carry):
        j0 = g * NBUF
        for b in range(NBUF):
            j = j0 + b
            gwait(j, b)
            scale(b)
            sfire(j, b)
        for b in range(NBUF):
            jn = j0 + NBUF + b

            @pl.when(jn < NCHUNK)
            def _():
                swait(j0 + b, b)
                gfire(jn, b)

        return carry

    lax.fori_loop(0, NROUND, round_body, 0)
    for b in range(NBUF):
        swait(NCHUNK - NBUF + b, b)


def kernel(x, table):
    idx = x.astype(jnp.int32).reshape(NW, NCHUNK, CHUNK)
    out = _emb_lookup(table, idx)
    return out.reshape(x.shape[0], x.shape[1], D)
